# Initial kernel scaffold; baseline (speedup 1.0000x reference)
import jax
import jax.numpy as jnp
from jax.experimental import pallas as pl


def kernel(p_t):
    raise NotImplementedError("write your pallas kernel here")


# speedup vs baseline: 2.7697x; 2.7697x over previous
import jax
import jax.numpy as jnp
import numpy as np
from jax.experimental import pallas as pl

_B = 16384
_R = 128
_C = 128

_KEY_HI = np.uint32(0)
_KEY_LO = np.uint32(42)


def _threefry2x32(x0, x1, k0, k1):
    ks2 = k0 ^ k1 ^ np.uint32(0x1BD11BDA)
    ks = (k0, k1, ks2)
    rot_a = (13, 15, 26, 6)
    rot_b = (17, 29, 16, 24)

    def rounds(x0, x1, rots):
        for r in rots:
            x0 = x0 + x1
            x1 = (x1 << np.uint32(r)) | (x1 >> np.uint32(32 - r))
            x1 = x1 ^ x0
        return x0, x1

    x0 = x0 + ks[0]
    x1 = x1 + ks[1]
    x0, x1 = rounds(x0, x1, rot_a)
    x0 = x0 + ks[1]
    x1 = x1 + ks[2] + np.uint32(1)
    x0, x1 = rounds(x0, x1, rot_b)
    x0 = x0 + ks[2]
    x1 = x1 + ks[0] + np.uint32(2)
    x0, x1 = rounds(x0, x1, rot_a)
    x0 = x0 + ks[0]
    x1 = x1 + ks[1] + np.uint32(3)
    x0, x1 = rounds(x0, x1, rot_b)
    x0 = x0 + ks[1]
    x1 = x1 + ks[2] + np.uint32(4)
    x0, x1 = rounds(x0, x1, rot_a)
    x0 = x0 + ks[2]
    x1 = x1 + ks[0] + np.uint32(5)
    return x0, x1


def _bits_to_gumbel(bits):
    tiny = np.float32(np.finfo(np.float32).tiny)
    mant = (bits >> np.uint32(9)) | np.uint32(0x3F800000)
    fl = jax.lax.bitcast_convert_type(mant, jnp.float32) - np.float32(1.0)
    u = jnp.maximum(tiny, fl * (np.float32(1.0) - tiny) + tiny)
    return -jnp.log(-jnp.log(u))


def _sample_body(p_ref, o_ref):
    p = p_ref[...]
    row = jax.lax.broadcasted_iota(jnp.uint32, (_R, _C), 0)
    col = jax.lax.broadcasted_iota(jnp.uint32, (_R, _C), 1)
    flat2 = (row * np.uint32(_C) + col) * np.uint32(2)

    a0, a1 = _threefry2x32(jnp.zeros_like(flat2), flat2, _KEY_HI, _KEY_LO)
    b0, b1 = _threefry2x32(
        jnp.zeros_like(flat2), flat2 + np.uint32(1), _KEY_HI, _KEY_LO
    )
    g0 = _bits_to_gumbel(a0 ^ a1)
    g1 = _bits_to_gumbel(b0 ^ b1)

    v0 = jnp.log(np.float32(1.0) - p) + g0
    v1 = jnp.log(p) + g1
    o_ref[...] = (v1 > v0).astype(jnp.float32)


def kernel(p_t):
    p2 = p_t.reshape(_R, _C)
    out = pl.pallas_call(
        _sample_body,
        out_shape=jax.ShapeDtypeStruct((_R, _C), jnp.float32),
    )(p2)
    return out.reshape(_B, 1, 1)
